# baseline (speedup 1.0000x reference)
import functools

import jax
import jax.numpy as jnp
from jax import lax
from jax.experimental import pallas as pl
from jax.experimental.pallas import tpu as pltpu
from jax.experimental.pallas import tpu_sc as plsc

NUM_USERS = 30000
NUM_ITEMS = 70000
NN = NUM_USERS + NUM_ITEMS
NPAD = 100096
EDGES = 3200000
D = 16
N_LAYERS = 3

NC = 2
NS = 16
NW = NC * NS

CHUNK = 128
UNIT = 4
UNITS_PER_W = 196
ROWS_PAD = NW * UNITS_PER_W * UNIT + 2 * UNIT
EDGES_PAD = ROWS_PAD * CHUNK
NODES_PER_TILE = NPAD // NS
MHALF = NPAD // NC
MROWS_PER_TILE = MHALF // NS


def _sc_layer_body(mode, *refs):
    if mode == "first":
        (ego_in, src_hbm, dst_hbm, val_hbm, part_hbm,
         src_g, dst_g, val_g, rows, acc,
         stage_sem, gather_sem, scatter_sem) = refs
    elif mode == "mid":
        (p_prev, src_hbm, dst_hbm, val_hbm, part_hbm, msum_out, ego_c,
         src_g, dst_g, val_g, rows, acc,
         stage_sem, gather_sem, scatter_sem) = refs
    else:
        (p_prev, msum_in, src_hbm, dst_hbm, val_hbm, part_hbm, msum_out,
         ego_c, src_g, dst_g, val_g, rows, acc,
         stage_sem, gather_sem, scatter_sem) = refs

    c = lax.axis_index("c")
    s = lax.axis_index("s")
    w = s * NC + c
    ub = w * UNITS_PER_W
    node_base = s * NODES_PER_TILE

    zero_idx = lax.iota(jnp.int32, 16) * 0

    def _vadd_rows(n):
        def _add(e, carry):
            rows[0, e] = rows[0, e] + rows[1, e]
            return carry

        lax.fori_loop(0, n, _add, 0)

    if mode == "first":
        ego_tbl = ego_in
    else:
        ego_tbl = ego_c.at[c]
        nfull = NODES_PER_TILE // CHUNK
        tail = NODES_PER_TILE - nfull * CHUNK

        def _comb(i, carry):
            base = node_base + i * CHUNK
            pltpu.sync_copy(p_prev.at[0, pl.ds(base, CHUNK)], rows.at[0])
            pltpu.sync_copy(p_prev.at[1, pl.ds(base, CHUNK)], rows.at[1])
            _vadd_rows(CHUNK)
            pltpu.sync_copy(rows.at[0], ego_c.at[c, pl.ds(base, CHUNK)])
            if mode == "mid":
                @pl.when(c == 0)
                def _():
                    pltpu.sync_copy(rows.at[0], msum_out.at[pl.ds(base, CHUNK)])
            return carry

        lax.fori_loop(0, nfull, _comb, 0)
        base_t = node_base + nfull * CHUNK
        pltpu.sync_copy(p_prev.at[0, pl.ds(base_t, tail)],
                        rows.at[0, pl.ds(0, tail)])
        pltpu.sync_copy(p_prev.at[1, pl.ds(base_t, tail)],
                        rows.at[1, pl.ds(0, tail)])
        _vadd_rows(tail)
        pltpu.sync_copy(rows.at[0, pl.ds(0, tail)],
                        ego_c.at[c, pl.ds(base_t, tail)])
        if mode == "mid":
            @pl.when(c == 0)
            def _():
                pltpu.sync_copy(rows.at[0, pl.ds(0, tail)],
                                msum_out.at[pl.ds(base_t, tail)])
        plsc.subcore_barrier()

        if mode == "last":
            mb = c * MHALF + s * MROWS_PER_TILE
            mfull = MROWS_PER_TILE // CHUNK
            mtail = MROWS_PER_TILE - mfull * CHUNK

            def _macc(i, carry):
                base = mb + i * CHUNK
                pltpu.sync_copy(ego_c.at[c, pl.ds(base, CHUNK)], rows.at[0])
                pltpu.sync_copy(msum_in.at[pl.ds(base, CHUNK)], rows.at[1])
                _vadd_rows(CHUNK)
                pltpu.sync_copy(rows.at[0], msum_out.at[pl.ds(base, CHUNK)])
                return carry

            lax.fori_loop(0, mfull, _macc, 0)
            mbase_t = mb + mfull * CHUNK
            pltpu.sync_copy(ego_c.at[c, pl.ds(mbase_t, mtail)],
                            rows.at[0, pl.ds(0, mtail)])
            pltpu.sync_copy(msum_in.at[pl.ds(mbase_t, mtail)],
                            rows.at[1, pl.ds(0, mtail)])
            _vadd_rows(mtail)
            pltpu.sync_copy(rows.at[0, pl.ds(0, mtail)],
                            msum_out.at[pl.ds(mbase_t, mtail)])

    def _chunk_proc(kr, ks):
        def _scale(g, carry):
            v16 = val_g[ks, pl.ds(g * 16, 16)]
            for jl in range(16):
                bj = v16.at[zero_idx + jl].get(mode="promise_in_bounds")
                e = g * 16 + jl
                rows[kr, e] = rows[kr, e] * bj
            return carry

        lax.fori_loop(0, CHUNK // 16, _scale, 0)
        for t in range(CHUNK // 16):
            pltpu.async_copy(rows.at[kr, pl.ds(t * 16, 16)],
                             acc.at[dst_g.at[ks, t]], scatter_sem,
                             add=True)

    def _stage(u, slot, issue=True):
        r0 = (ub + u) * UNIT
        k0 = slot * UNIT
        mk = pltpu.async_copy if issue else pltpu.make_async_copy
        return (
            mk(src_hbm.at[pl.ds(r0, UNIT)],
               src_g.at[pl.ds(k0, UNIT)], stage_sem),
            mk(dst_hbm.at[pl.ds(r0, UNIT)],
               dst_g.at[pl.ds(k0, UNIT)], stage_sem),
            mk(val_hbm.at[pl.ds(r0, UNIT)],
               val_g.at[pl.ds(k0, UNIT)], stage_sem),
        )

    def _issue_gathers(rslot, sslot):
        return [pltpu.async_copy(ego_tbl.at[src_g.at[sslot * UNIT + j]],
                                 rows.at[rslot * UNIT + j], gather_sem)
                for j in range(UNIT)]

    def _drain_scatters(rslot):
        for j in range(UNIT):
            pltpu.make_async_copy(part_hbm.at[0, pl.ds(0, CHUNK)],
                                  rows.at[rslot * UNIT + j],
                                  scatter_sem).wait()

    def _process_unit(u, h):
        rslot = h % 2
        nxt = 1 - rslot
        if h == 0:
            @pl.when(u > 0)
            def _():
                _drain_scatters(nxt)
        else:
            _drain_scatters(nxt)
        for d in _stage(u + 1, (h + 1) % 4, issue=False):
            d.wait()
        gds = _issue_gathers(nxt, (h + 1) % 4)
        _stage(u + 2, (h + 2) % 4)
        for j in range(UNIT):
            _chunk_proc(rslot * UNIT + j, h * UNIT + j)
        for d in gds:
            d.wait()

    zero_row = jnp.zeros((D,), jnp.float32)

    def _zrow(i, carry):
        rows[0, i] = zero_row
        return carry

    lax.fori_loop(0, CHUNK, _zrow, 0)
    nfull_z = NODES_PER_TILE // CHUNK
    for k in range(nfull_z):
        pltpu.sync_copy(rows.at[0], acc.at[pl.ds(node_base + k * CHUNK, CHUNK)])
    tail_z = NODES_PER_TILE - nfull_z * CHUNK
    if tail_z:
        pltpu.sync_copy(rows.at[0, pl.ds(0, tail_z)],
                        acc.at[pl.ds(node_base + nfull_z * CHUNK, tail_z)])
    plsc.subcore_barrier()

    for d in _stage(0, 0):
        d.wait()
    for d in _issue_gathers(0, 0):
        d.wait()
    _stage(1, 1)

    def _quad(i, carry):
        for h in range(4):
            _process_unit(4 * i + h, h)
        return carry

    lax.fori_loop(0, UNITS_PER_W // 4, _quad, 0)
    _drain_scatters(1)
    for d in _stage(UNITS_PER_W + 1, 1, issue=False):
        d.wait()
    plsc.subcore_barrier()

    pltpu.sync_copy(acc.at[pl.ds(node_base, NODES_PER_TILE)],
                    part_hbm.at[c, pl.ds(node_base, NODES_PER_TILE)])


_PARTS_T = jax.ShapeDtypeStruct((NC, NPAD, D), jnp.float32)
_MSUM_T = jax.ShapeDtypeStruct((NPAD, D), jnp.float32)
_MESH = plsc.VectorSubcoreMesh(core_axis_name="c", subcore_axis_name="s")
_SCRATCH = [
    pltpu.VMEM((4 * UNIT, CHUNK), jnp.int32),
    pltpu.VMEM((4 * UNIT, CHUNK // 16, 16), jnp.int32),
    pltpu.VMEM((4 * UNIT, CHUNK), jnp.float32),
    pltpu.VMEM((2 * UNIT, CHUNK, D), jnp.float32),
    pltpu.VMEM_SHARED((NPAD, D), jnp.float32),
    pltpu.SemaphoreType.DMA,
    pltpu.SemaphoreType.DMA,
    pltpu.SemaphoreType.DMA,
]


def _mk_sc(mode, outs):
    return pl.kernel(
        functools.partial(_sc_layer_body, mode),
        out_type=outs,
        mesh=_MESH,
        compiler_params=pltpu.CompilerParams(use_tc_tiling_on_sc=False),
        scratch_types=_SCRATCH,
    )


_sc_first = _mk_sc("first", _PARTS_T)
_sc_mid = _mk_sc("mid", (_PARTS_T, _MSUM_T, _PARTS_T))
_sc_last = _mk_sc("last", (_PARTS_T, _MSUM_T, _PARTS_T))


_CW = 128
_CR = NPAD * D // _CW


def _final_body(p_ref, m_ref, out_ref):
    out_ref[...] = (m_ref[...] + p_ref[0] + p_ref[1]) * (1.0 / N_LAYERS)


def _final(parts, msum):
    p = parts.reshape(NC, _CR, _CW)
    m = msum.reshape(_CR, _CW)
    out = pl.pallas_call(
        _final_body,
        out_shape=jax.ShapeDtypeStruct((_CR, _CW), jnp.float32),
    )(p, m)
    return out.reshape(NPAD, D)


def kernel(user_emb, item_emb, adj_vals, edge_src, edge_dst):
    ego = jnp.concatenate(
        [user_emb, item_emb,
         jnp.zeros((NPAD - NN, D), jnp.float32)], axis=0)

    npad_e = EDGES_PAD - EDGES
    src_p = jnp.concatenate(
        [edge_src.astype(jnp.int32), jnp.zeros((npad_e,), jnp.int32)])
    dst_p = jnp.concatenate(
        [edge_dst.astype(jnp.int32),
         NN + (jnp.arange(npad_e, dtype=jnp.int32) % (NPAD - NN))])
    val_p = jnp.concatenate([adj_vals, jnp.zeros((npad_e,), jnp.float32)])

    src2 = src_p.reshape(ROWS_PAD, CHUNK)
    dst3 = dst_p.reshape(ROWS_PAD, CHUNK // 16, 16)
    val2 = val_p.reshape(ROWS_PAD, CHUNK)

    parts = _sc_first(ego, src2, dst3, val2)
    parts, msum, _ = _sc_mid(parts, src2, dst3, val2)
    parts, msum, _ = _sc_last(parts, msum, src2, dst3, val2)

    final = _final(parts, msum)
    return final[:NUM_USERS], final[NUM_USERS:NN]


# speedup vs baseline: 1.1976x; 1.1976x over previous
import functools

import jax
import jax.numpy as jnp
from jax import lax
from jax.experimental import pallas as pl
from jax.experimental.pallas import tpu as pltpu
from jax.experimental.pallas import tpu_sc as plsc

NUM_USERS = 30000
NUM_ITEMS = 70000
NN = NUM_USERS + NUM_ITEMS
NPAD = 100096
EDGES = 3200000
D = 16
N_LAYERS = 3

NC = 2
NS = 16
NW = NC * NS

CHUNK = 128
UNIT = 4
UNITS_PER_W = 196
ROWS_PAD = NW * UNITS_PER_W * UNIT + 2 * UNIT
EDGES_PAD = ROWS_PAD * CHUNK
NODES_PER_TILE = NPAD // NS
MHALF = NPAD // NC
MROWS_PER_TILE = MHALF // NS


def _sc_layer_body(ego_hbm, src_hbm, dst_hbm, val_hbm, part_hbm,
                   src_g, dst_g, val_g, rows, acc,
                   stage_sem, gather_sem, scatter_sem):
    c = lax.axis_index("c")
    s = lax.axis_index("s")
    w = s * NC + c
    ub = w * UNITS_PER_W
    node_base = s * NODES_PER_TILE

    zero_idx = lax.iota(jnp.int32, 16) * 0

    def _chunk_proc(kr, ks):
        def _scale(g, carry):
            v16 = val_g[ks, pl.ds(g * 16, 16)]
            for jl in range(16):
                bj = v16.at[zero_idx + jl].get(mode="promise_in_bounds")
                e = g * 16 + jl
                rows[kr, e] = rows[kr, e] * bj
            return carry

        lax.fori_loop(0, CHUNK // 16, _scale, 0)
        for t in range(CHUNK // 16):
            pltpu.async_copy(rows.at[kr, pl.ds(t * 16, 16)],
                             acc.at[dst_g.at[ks, t]], scatter_sem,
                             add=True)

    def _stage(u, slot, issue=True):
        r0 = (ub + u) * UNIT
        k0 = slot * UNIT
        mk = pltpu.async_copy if issue else pltpu.make_async_copy
        return (
            mk(src_hbm.at[pl.ds(r0, UNIT)],
               src_g.at[pl.ds(k0, UNIT)], stage_sem),
            mk(dst_hbm.at[pl.ds(r0, UNIT)],
               dst_g.at[pl.ds(k0, UNIT)], stage_sem),
            mk(val_hbm.at[pl.ds(r0, UNIT)],
               val_g.at[pl.ds(k0, UNIT)], stage_sem),
        )

    def _issue_gathers(rslot, sslot):
        return [pltpu.async_copy(ego_hbm.at[src_g.at[sslot * UNIT + j]],
                                 rows.at[rslot * UNIT + j], gather_sem)
                for j in range(UNIT)]

    def _drain_scatters(rslot):
        for j in range(UNIT):
            pltpu.make_async_copy(ego_hbm.at[pl.ds(0, CHUNK)],
                                  rows.at[rslot * UNIT + j],
                                  scatter_sem).wait()

    def _process_unit(u, h):
        rslot = h % 2
        nxt = 1 - rslot
        if h == 0:
            @pl.when(u > 0)
            def _():
                _drain_scatters(nxt)
        else:
            _drain_scatters(nxt)
        for d in _stage(u + 1, (h + 1) % 4, issue=False):
            d.wait()
        gds = _issue_gathers(nxt, (h + 1) % 4)
        _stage(u + 2, (h + 2) % 4)
        for j in range(UNIT):
            _chunk_proc(rslot * UNIT + j, h * UNIT + j)
        for d in gds:
            d.wait()

    zero_row = jnp.zeros((D,), jnp.float32)

    def _zrow(i, carry):
        rows[0, i] = zero_row
        return carry

    lax.fori_loop(0, CHUNK, _zrow, 0)
    nfull_z = NODES_PER_TILE // CHUNK
    for k in range(nfull_z):
        pltpu.sync_copy(rows.at[0], acc.at[pl.ds(node_base + k * CHUNK, CHUNK)])
    tail_z = NODES_PER_TILE - nfull_z * CHUNK
    if tail_z:
        pltpu.sync_copy(rows.at[0, pl.ds(0, tail_z)],
                        acc.at[pl.ds(node_base + nfull_z * CHUNK, tail_z)])
    plsc.subcore_barrier()

    for d in _stage(0, 0):
        d.wait()
    for d in _issue_gathers(0, 0):
        d.wait()
    _stage(1, 1)

    def _quad(i, carry):
        for h in range(4):
            _process_unit(4 * i + h, h)
        return carry

    lax.fori_loop(0, UNITS_PER_W // 4, _quad, 0)
    _drain_scatters(1)
    for d in _stage(UNITS_PER_W + 1, 1, issue=False):
        d.wait()
    plsc.subcore_barrier()

    pltpu.sync_copy(acc.at[pl.ds(node_base, NODES_PER_TILE)],
                    part_hbm.at[c, pl.ds(node_base, NODES_PER_TILE)])


_sc_layer = pl.kernel(
    _sc_layer_body,
    out_type=jax.ShapeDtypeStruct((NC, NPAD, D), jnp.float32),
    mesh=plsc.VectorSubcoreMesh(core_axis_name="c", subcore_axis_name="s"),
    compiler_params=pltpu.CompilerParams(use_tc_tiling_on_sc=False),
    scratch_types=[
        pltpu.VMEM((4 * UNIT, CHUNK), jnp.int32),
        pltpu.VMEM((4 * UNIT, CHUNK // 16, 16), jnp.int32),
        pltpu.VMEM((4 * UNIT, CHUNK), jnp.float32),
        pltpu.VMEM((2 * UNIT, CHUNK, D), jnp.float32),
        pltpu.VMEM_SHARED((NPAD, D), jnp.float32),
        pltpu.SemaphoreType.DMA,
        pltpu.SemaphoreType.DMA,
        pltpu.SemaphoreType.DMA,
    ],
)


_CW = 128
_CR = NPAD * D // _CW


def _combine_body(last, p_ref, m_ref, ego_ref, mout_ref):
    e = p_ref[0] + p_ref[1]
    ego_ref[...] = e
    if last:
        mout_ref[...] = (m_ref[...] + e) * (1.0 / N_LAYERS)
    else:
        mout_ref[...] = m_ref[...] + e


def _combine(parts, msum, last):
    p = parts.reshape(NC, _CR, _CW)
    ego, mout = pl.pallas_call(
        functools.partial(_combine_body, last),
        out_shape=[
            jax.ShapeDtypeStruct((_CR, _CW), jnp.float32),
            jax.ShapeDtypeStruct((_CR, _CW), jnp.float32),
        ],
    )(p, msum)
    return ego.reshape(NPAD, D), mout


def kernel(user_emb, item_emb, adj_vals, edge_src, edge_dst):
    ego = jnp.concatenate(
        [user_emb, item_emb,
         jnp.zeros((NPAD - NN, D), jnp.float32)], axis=0)

    npad_e = EDGES_PAD - EDGES
    src_p = jnp.concatenate(
        [edge_src.astype(jnp.int32), jnp.zeros((npad_e,), jnp.int32)])
    dst_p = jnp.concatenate(
        [edge_dst.astype(jnp.int32),
         NN + (jnp.arange(npad_e, dtype=jnp.int32) % (NPAD - NN))])
    val_p = jnp.concatenate([adj_vals, jnp.zeros((npad_e,), jnp.float32)])

    src2 = src_p.reshape(ROWS_PAD, CHUNK)
    dst3 = dst_p.reshape(ROWS_PAD, CHUNK // 16, 16)
    val2 = val_p.reshape(ROWS_PAD, CHUNK)

    msum = jnp.zeros((_CR, _CW), jnp.float32)
    for layer in range(N_LAYERS):
        parts = _sc_layer(ego, src2, dst3, val2)
        ego, msum = _combine(parts, msum, last=(layer == N_LAYERS - 1))

    final = msum.reshape(NPAD, D)
    return final[:NUM_USERS], final[NUM_USERS:NN]


# speedup vs baseline: 1.1987x; 1.0009x over previous
import functools

import jax
import jax.numpy as jnp
from jax import lax
from jax.experimental import pallas as pl
from jax.experimental.pallas import tpu as pltpu
from jax.experimental.pallas import tpu_sc as plsc

NUM_USERS = 30000
NUM_ITEMS = 70000
NN = NUM_USERS + NUM_ITEMS
NPAD = 100096
EDGES = 3200000
D = 16
N_LAYERS = 3

NC = 2
NS = 16
NW = NC * NS

CHUNK = 128
UNIT = 4
UNITS_PER_W = 196
ROWS_PAD = NW * UNITS_PER_W * UNIT + 2 * UNIT
EDGES_PAD = ROWS_PAD * CHUNK
NODES_PER_TILE = NPAD // NS
MHALF = NPAD // NC
MROWS_PER_TILE = MHALF // NS


def _sc_layer_body(ego_hbm, src_hbm, dst_hbm, val_hbm, part_hbm,
                   src_g, dst_g, val_g, rows, acc,
                   stage_sem, gather_sem, scatter_sem):
    c = lax.axis_index("c")
    s = lax.axis_index("s")
    w = s * NC + c
    ub = w * UNITS_PER_W
    node_base = s * NODES_PER_TILE

    zero_idx = lax.iota(jnp.int32, 16) * 0

    def _chunk_proc(kr, ks):
        def _scale(g, carry):
            v16 = val_g[ks, pl.ds(g * 16, 16)]
            for jl in range(16):
                bj = v16.at[zero_idx + jl].get(mode="promise_in_bounds")
                e = g * 16 + jl
                rows[kr, e] = rows[kr, e] * bj
            return carry

        lax.fori_loop(0, CHUNK // 16, _scale, 0)
        for t in range(CHUNK // 32):
            pltpu.async_copy(rows.at[kr, pl.ds(t * 32, 32)],
                             acc.at[dst_g.at[ks, t]], scatter_sem,
                             add=True)

    def _stage(u, slot, issue=True):
        r0 = (ub + u) * UNIT
        k0 = slot * UNIT
        mk = pltpu.async_copy if issue else pltpu.make_async_copy
        return (
            mk(src_hbm.at[pl.ds(r0, UNIT)],
               src_g.at[pl.ds(k0, UNIT)], stage_sem),
            mk(dst_hbm.at[pl.ds(r0, UNIT)],
               dst_g.at[pl.ds(k0, UNIT)], stage_sem),
            mk(val_hbm.at[pl.ds(r0, UNIT)],
               val_g.at[pl.ds(k0, UNIT)], stage_sem),
        )

    def _issue_gathers(rslot, sslot):
        return [pltpu.async_copy(ego_hbm.at[src_g.at[sslot * UNIT + j]],
                                 rows.at[rslot * UNIT + j], gather_sem)
                for j in range(UNIT)]

    def _drain_scatters(rslot):
        for j in range(UNIT):
            pltpu.make_async_copy(ego_hbm.at[pl.ds(0, CHUNK)],
                                  rows.at[rslot * UNIT + j],
                                  scatter_sem).wait()

    def _process_unit(u, h):
        rslot = h % 2
        nxt = 1 - rslot
        if h == 0:
            @pl.when(u > 0)
            def _():
                _drain_scatters(nxt)
        else:
            _drain_scatters(nxt)
        for d in _stage(u + 1, (h + 1) % 4, issue=False):
            d.wait()
        gds = _issue_gathers(nxt, (h + 1) % 4)
        _stage(u + 2, (h + 2) % 4)
        for j in range(UNIT):
            _chunk_proc(rslot * UNIT + j, h * UNIT + j)
        for d in gds:
            d.wait()

    zero_row = jnp.zeros((D,), jnp.float32)

    def _zrow(i, carry):
        rows[0, i] = zero_row
        return carry

    lax.fori_loop(0, CHUNK, _zrow, 0)
    nfull_z = NODES_PER_TILE // CHUNK
    for k in range(nfull_z):
        pltpu.sync_copy(rows.at[0], acc.at[pl.ds(node_base + k * CHUNK, CHUNK)])
    tail_z = NODES_PER_TILE - nfull_z * CHUNK
    if tail_z:
        pltpu.sync_copy(rows.at[0, pl.ds(0, tail_z)],
                        acc.at[pl.ds(node_base + nfull_z * CHUNK, tail_z)])
    plsc.subcore_barrier()

    for d in _stage(0, 0):
        d.wait()
    for d in _issue_gathers(0, 0):
        d.wait()
    _stage(1, 1)

    def _quad(i, carry):
        for h in range(4):
            _process_unit(4 * i + h, h)
        return carry

    lax.fori_loop(0, UNITS_PER_W // 4, _quad, 0)
    _drain_scatters(1)
    for d in _stage(UNITS_PER_W + 1, 1, issue=False):
        d.wait()
    plsc.subcore_barrier()

    pltpu.sync_copy(acc.at[pl.ds(node_base, NODES_PER_TILE)],
                    part_hbm.at[c, pl.ds(node_base, NODES_PER_TILE)])


_sc_layer = pl.kernel(
    _sc_layer_body,
    out_type=jax.ShapeDtypeStruct((NC, NPAD, D), jnp.float32),
    mesh=plsc.VectorSubcoreMesh(core_axis_name="c", subcore_axis_name="s"),
    compiler_params=pltpu.CompilerParams(use_tc_tiling_on_sc=False),
    scratch_types=[
        pltpu.VMEM((4 * UNIT, CHUNK), jnp.int32),
        pltpu.VMEM((4 * UNIT, CHUNK // 32, 32), jnp.int32),
        pltpu.VMEM((4 * UNIT, CHUNK), jnp.float32),
        pltpu.VMEM((2 * UNIT, CHUNK, D), jnp.float32),
        pltpu.VMEM_SHARED((NPAD, D), jnp.float32),
        pltpu.SemaphoreType.DMA,
        pltpu.SemaphoreType.DMA,
        pltpu.SemaphoreType.DMA,
    ],
)


_CW = 128
_CR = NPAD * D // _CW


def _combine_body(last, p_ref, m_ref, ego_ref, mout_ref):
    e = p_ref[0] + p_ref[1]
    ego_ref[...] = e
    if last:
        mout_ref[...] = (m_ref[...] + e) * (1.0 / N_LAYERS)
    else:
        mout_ref[...] = m_ref[...] + e


def _combine(parts, msum, last):
    p = parts.reshape(NC, _CR, _CW)
    ego, mout = pl.pallas_call(
        functools.partial(_combine_body, last),
        out_shape=[
            jax.ShapeDtypeStruct((_CR, _CW), jnp.float32),
            jax.ShapeDtypeStruct((_CR, _CW), jnp.float32),
        ],
    )(p, msum)
    return ego.reshape(NPAD, D), mout


def kernel(user_emb, item_emb, adj_vals, edge_src, edge_dst):
    ego = jnp.concatenate(
        [user_emb, item_emb,
         jnp.zeros((NPAD - NN, D), jnp.float32)], axis=0)

    npad_e = EDGES_PAD - EDGES
    src_p = jnp.concatenate(
        [edge_src.astype(jnp.int32), jnp.zeros((npad_e,), jnp.int32)])
    dst_p = jnp.concatenate(
        [edge_dst.astype(jnp.int32),
         NN + (jnp.arange(npad_e, dtype=jnp.int32) % (NPAD - NN))])
    val_p = jnp.concatenate([adj_vals, jnp.zeros((npad_e,), jnp.float32)])

    src2 = src_p.reshape(ROWS_PAD, CHUNK)
    dst3 = dst_p.reshape(ROWS_PAD, CHUNK // 32, 32)
    val2 = val_p.reshape(ROWS_PAD, CHUNK)

    msum = jnp.zeros((_CR, _CW), jnp.float32)
    for layer in range(N_LAYERS):
        parts = _sc_layer(ego, src2, dst3, val2)
        ego, msum = _combine(parts, msum, last=(layer == N_LAYERS - 1))

    final = msum.reshape(NPAD, D)
    return final[:NUM_USERS], final[NUM_USERS:NN]


# speedup vs baseline: 1.2243x; 1.0214x over previous
import functools

import jax
import jax.numpy as jnp
from jax import lax
from jax.experimental import pallas as pl
from jax.experimental.pallas import tpu as pltpu
from jax.experimental.pallas import tpu_sc as plsc

NUM_USERS = 30000
NUM_ITEMS = 70000
NN = NUM_USERS + NUM_ITEMS
NPAD = 100096
EDGES = 3200000
D = 16
N_LAYERS = 3

NC = 2
NS = 16
NW = NC * NS

CHUNK = 128
UNIT = 4
UNITS_PER_W = 196
UNITS_C0 = 204
UNITS_C1 = 188
ROWS_PAD = NS * (UNITS_C0 + UNITS_C1) * UNIT + 2 * UNIT
EDGES_PAD = ROWS_PAD * CHUNK
NODES_PER_TILE = NPAD // NS
MHALF = NPAD // NC
MROWS_PER_TILE = MHALF // NS


def _sc_layer_body(ego_hbm, src_hbm, dst_hbm, val_hbm, part_hbm,
                   src_g, dst_g, val_g, rows, acc,
                   stage_sem, gather_sem, scatter_sem):
    c = lax.axis_index("c")
    s = lax.axis_index("s")
    ub = jnp.where(c == 0, s * UNITS_C0, NS * UNITS_C0 + s * UNITS_C1)
    u_w = jnp.where(c == 0, UNITS_C0, UNITS_C1)
    node_base = s * NODES_PER_TILE

    zero_idx = lax.iota(jnp.int32, 16) * 0

    def _chunk_proc(kr, ks):
        def _scale(g, carry):
            v16 = val_g[ks, pl.ds(g * 16, 16)]
            for jl in range(16):
                bj = v16.at[zero_idx + jl].get(mode="promise_in_bounds")
                e = g * 16 + jl
                rows[kr, e] = rows[kr, e] * bj
            return carry

        lax.fori_loop(0, CHUNK // 16, _scale, 0)
        for t in range(CHUNK // 16):
            pltpu.async_copy(rows.at[kr, pl.ds(t * 16, 16)],
                             acc.at[dst_g.at[ks, t]], scatter_sem,
                             add=True)

    def _stage(u, slot, issue=True):
        r0 = (ub + u) * UNIT
        k0 = slot * UNIT
        mk = pltpu.async_copy if issue else pltpu.make_async_copy
        return (
            mk(src_hbm.at[pl.ds(r0, UNIT)],
               src_g.at[pl.ds(k0, UNIT)], stage_sem),
            mk(dst_hbm.at[pl.ds(r0, UNIT)],
               dst_g.at[pl.ds(k0, UNIT)], stage_sem),
            mk(val_hbm.at[pl.ds(r0, UNIT)],
               val_g.at[pl.ds(k0, UNIT)], stage_sem),
        )

    def _issue_gathers(rslot, sslot):
        return [pltpu.async_copy(ego_hbm.at[src_g.at[sslot * UNIT + j]],
                                 rows.at[rslot * UNIT + j], gather_sem)
                for j in range(UNIT)]

    def _drain_scatters(rslot):
        for j in range(UNIT):
            pltpu.make_async_copy(ego_hbm.at[pl.ds(0, CHUNK)],
                                  rows.at[rslot * UNIT + j],
                                  scatter_sem).wait()

    def _process_unit(u, h):
        rslot = h % 2
        nxt = 1 - rslot
        if h == 0:
            @pl.when(u > 0)
            def _():
                _drain_scatters(nxt)
        else:
            _drain_scatters(nxt)
        for d in _stage(u + 1, (h + 1) % 4, issue=False):
            d.wait()
        gds = _issue_gathers(nxt, (h + 1) % 4)
        _stage(u + 2, (h + 2) % 4)
        for j in range(UNIT):
            _chunk_proc(rslot * UNIT + j, h * UNIT + j)
        for d in gds:
            d.wait()

    zero_row = jnp.zeros((D,), jnp.float32)

    def _zrow(i, carry):
        rows[0, i] = zero_row
        return carry

    lax.fori_loop(0, CHUNK, _zrow, 0)
    nfull_z = NODES_PER_TILE // CHUNK
    for k in range(nfull_z):
        pltpu.sync_copy(rows.at[0], acc.at[pl.ds(node_base + k * CHUNK, CHUNK)])
    tail_z = NODES_PER_TILE - nfull_z * CHUNK
    if tail_z:
        pltpu.sync_copy(rows.at[0, pl.ds(0, tail_z)],
                        acc.at[pl.ds(node_base + nfull_z * CHUNK, tail_z)])
    plsc.subcore_barrier()

    for d in _stage(0, 0):
        d.wait()
    for d in _issue_gathers(0, 0):
        d.wait()
    _stage(1, 1)

    def _quad(i, carry):
        for h in range(4):
            _process_unit(4 * i + h, h)
        return carry

    lax.fori_loop(0, u_w // 4, _quad, 0)
    _drain_scatters(1)
    for d in _stage(u_w + 1, 1, issue=False):
        d.wait()
    plsc.subcore_barrier()

    pltpu.sync_copy(acc.at[pl.ds(node_base, NODES_PER_TILE)],
                    part_hbm.at[c, pl.ds(node_base, NODES_PER_TILE)])


_sc_layer = pl.kernel(
    _sc_layer_body,
    out_type=jax.ShapeDtypeStruct((NC, NPAD, D), jnp.float32),
    mesh=plsc.VectorSubcoreMesh(core_axis_name="c", subcore_axis_name="s"),
    compiler_params=pltpu.CompilerParams(use_tc_tiling_on_sc=False),
    scratch_types=[
        pltpu.VMEM((4 * UNIT, CHUNK), jnp.int32),
        pltpu.VMEM((4 * UNIT, CHUNK // 16, 16), jnp.int32),
        pltpu.VMEM((4 * UNIT, CHUNK), jnp.float32),
        pltpu.VMEM((2 * UNIT, CHUNK, D), jnp.float32),
        pltpu.VMEM_SHARED((NPAD, D), jnp.float32),
        pltpu.SemaphoreType.DMA,
        pltpu.SemaphoreType.DMA,
        pltpu.SemaphoreType.DMA,
    ],
)


_CW = 128
_CR = NPAD * D // _CW


def _combine_body(last, p_ref, m_ref, ego_ref, mout_ref):
    e = p_ref[0] + p_ref[1]
    ego_ref[...] = e
    if last:
        mout_ref[...] = (m_ref[...] + e) * (1.0 / N_LAYERS)
    else:
        mout_ref[...] = m_ref[...] + e


def _combine(parts, msum, last):
    p = parts.reshape(NC, _CR, _CW)
    ego, mout = pl.pallas_call(
        functools.partial(_combine_body, last),
        out_shape=[
            jax.ShapeDtypeStruct((_CR, _CW), jnp.float32),
            jax.ShapeDtypeStruct((_CR, _CW), jnp.float32),
        ],
    )(p, msum)
    return ego.reshape(NPAD, D), mout


def kernel(user_emb, item_emb, adj_vals, edge_src, edge_dst):
    ego = jnp.concatenate(
        [user_emb, item_emb,
         jnp.zeros((NPAD - NN, D), jnp.float32)], axis=0)

    npad_e = EDGES_PAD - EDGES
    src_p = jnp.concatenate(
        [edge_src.astype(jnp.int32), jnp.zeros((npad_e,), jnp.int32)])
    dst_p = jnp.concatenate(
        [edge_dst.astype(jnp.int32),
         NN + (jnp.arange(npad_e, dtype=jnp.int32) % (NPAD - NN))])
    val_p = jnp.concatenate([adj_vals, jnp.zeros((npad_e,), jnp.float32)])

    src2 = src_p.reshape(ROWS_PAD, CHUNK)
    dst3 = dst_p.reshape(ROWS_PAD, CHUNK // 16, 16)
    val2 = val_p.reshape(ROWS_PAD, CHUNK)

    msum = jnp.zeros((_CR, _CW), jnp.float32)
    for layer in range(N_LAYERS):
        parts = _sc_layer(ego, src2, dst3, val2)
        ego, msum = _combine(parts, msum, last=(layer == N_LAYERS - 1))

    final = msum.reshape(NPAD, D)
    return final[:NUM_USERS], final[NUM_USERS:NN]


# speedup vs baseline: 1.2567x; 1.0264x over previous
import functools

import jax
import jax.numpy as jnp
from jax import lax
from jax.experimental import pallas as pl
from jax.experimental.pallas import tpu as pltpu
from jax.experimental.pallas import tpu_sc as plsc

NUM_USERS = 30000
NUM_ITEMS = 70000
NN = NUM_USERS + NUM_ITEMS
NPAD = 100096
EDGES = 3200000
D = 16
N_LAYERS = 3

NC = 2
NS = 16
NW = NC * NS

CHUNK = 128
UNIT = 4
UNITS_PER_W = 196
UNITS_C0 = 216
UNITS_C1 = 176
ROWS_PAD = NS * (UNITS_C0 + UNITS_C1) * UNIT + 2 * UNIT
EDGES_PAD = ROWS_PAD * CHUNK
NODES_PER_TILE = NPAD // NS
MHALF = NPAD // NC
MROWS_PER_TILE = MHALF // NS


def _sc_layer_body(ego_hbm, src_hbm, dst_hbm, val_hbm, part_hbm,
                   src_g, dst_g, val_g, rows, acc,
                   stage_sem, gather_sem, scatter_sem):
    c = lax.axis_index("c")
    s = lax.axis_index("s")
    ub = jnp.where(c == 0, s * UNITS_C0, NS * UNITS_C0 + s * UNITS_C1)
    u_w = jnp.where(c == 0, UNITS_C0, UNITS_C1)
    node_base = s * NODES_PER_TILE

    zero_idx = lax.iota(jnp.int32, 16) * 0

    def _chunk_proc(kr, ks):
        def _scale(g, carry):
            v16 = val_g[ks, pl.ds(g * 16, 16)]
            for jl in range(16):
                bj = v16.at[zero_idx + jl].get(mode="promise_in_bounds")
                e = g * 16 + jl
                rows[kr, e] = rows[kr, e] * bj
            return carry

        lax.fori_loop(0, CHUNK // 16, _scale, 0)
        for t in range(CHUNK // 16):
            pltpu.async_copy(rows.at[kr, pl.ds(t * 16, 16)],
                             acc.at[dst_g.at[ks, t]], scatter_sem,
                             add=True)

    def _stage(u, slot, issue=True):
        r0 = (ub + u) * UNIT
        k0 = slot * UNIT
        mk = pltpu.async_copy if issue else pltpu.make_async_copy
        return (
            mk(src_hbm.at[pl.ds(r0, UNIT)],
               src_g.at[pl.ds(k0, UNIT)], stage_sem),
            mk(dst_hbm.at[pl.ds(r0, UNIT)],
               dst_g.at[pl.ds(k0, UNIT)], stage_sem),
            mk(val_hbm.at[pl.ds(r0, UNIT)],
               val_g.at[pl.ds(k0, UNIT)], stage_sem),
        )

    def _issue_gathers(rslot, sslot):
        return [pltpu.async_copy(ego_hbm.at[src_g.at[sslot * UNIT + j]],
                                 rows.at[rslot * UNIT + j], gather_sem)
                for j in range(UNIT)]

    def _drain_scatters(rslot):
        for j in range(UNIT):
            pltpu.make_async_copy(ego_hbm.at[pl.ds(0, CHUNK)],
                                  rows.at[rslot * UNIT + j],
                                  scatter_sem).wait()

    def _process_unit(u, h):
        rslot = h % 2
        nxt = 1 - rslot
        if h == 0:
            @pl.when(u > 0)
            def _():
                _drain_scatters(nxt)
        else:
            _drain_scatters(nxt)
        for d in _stage(u + 1, (h + 1) % 4, issue=False):
            d.wait()
        gds = _issue_gathers(nxt, (h + 1) % 4)
        _stage(u + 2, (h + 2) % 4)
        for j in range(UNIT):
            _chunk_proc(rslot * UNIT + j, h * UNIT + j)
        for d in gds:
            d.wait()

    zero_row = jnp.zeros((D,), jnp.float32)

    def _zrow(i, carry):
        rows[0, i] = zero_row
        return carry

    lax.fori_loop(0, CHUNK, _zrow, 0)
    nfull_z = NODES_PER_TILE // CHUNK
    for k in range(nfull_z):
        pltpu.sync_copy(rows.at[0], acc.at[pl.ds(node_base + k * CHUNK, CHUNK)])
    tail_z = NODES_PER_TILE - nfull_z * CHUNK
    if tail_z:
        pltpu.sync_copy(rows.at[0, pl.ds(0, tail_z)],
                        acc.at[pl.ds(node_base + nfull_z * CHUNK, tail_z)])
    plsc.subcore_barrier()

    for d in _stage(0, 0):
        d.wait()
    for d in _issue_gathers(0, 0):
        d.wait()
    _stage(1, 1)

    def _quad(i, carry):
        for h in range(4):
            _process_unit(4 * i + h, h)
        return carry

    lax.fori_loop(0, u_w // 4, _quad, 0)
    _drain_scatters(1)
    for d in _stage(u_w + 1, 1, issue=False):
        d.wait()
    plsc.subcore_barrier()

    pltpu.sync_copy(acc.at[pl.ds(node_base, NODES_PER_TILE)],
                    part_hbm.at[c, pl.ds(node_base, NODES_PER_TILE)])


_sc_layer = pl.kernel(
    _sc_layer_body,
    out_type=jax.ShapeDtypeStruct((NC, NPAD, D), jnp.float32),
    mesh=plsc.VectorSubcoreMesh(core_axis_name="c", subcore_axis_name="s"),
    compiler_params=pltpu.CompilerParams(use_tc_tiling_on_sc=False),
    scratch_types=[
        pltpu.VMEM((4 * UNIT, CHUNK), jnp.int32),
        pltpu.VMEM((4 * UNIT, CHUNK // 16, 16), jnp.int32),
        pltpu.VMEM((4 * UNIT, CHUNK), jnp.float32),
        pltpu.VMEM((2 * UNIT, CHUNK, D), jnp.float32),
        pltpu.VMEM_SHARED((NPAD, D), jnp.float32),
        pltpu.SemaphoreType.DMA,
        pltpu.SemaphoreType.DMA,
        pltpu.SemaphoreType.DMA,
    ],
)


_CW = 128
_CR = NPAD * D // _CW


def _combine_body(last, p_ref, m_ref, ego_ref, mout_ref):
    e = p_ref[0] + p_ref[1]
    ego_ref[...] = e
    if last:
        mout_ref[...] = (m_ref[...] + e) * (1.0 / N_LAYERS)
    else:
        mout_ref[...] = m_ref[...] + e


def _combine(parts, msum, last):
    p = parts.reshape(NC, _CR, _CW)
    ego, mout = pl.pallas_call(
        functools.partial(_combine_body, last),
        out_shape=[
            jax.ShapeDtypeStruct((_CR, _CW), jnp.float32),
            jax.ShapeDtypeStruct((_CR, _CW), jnp.float32),
        ],
    )(p, msum)
    return ego.reshape(NPAD, D), mout


def kernel(user_emb, item_emb, adj_vals, edge_src, edge_dst):
    ego = jnp.concatenate(
        [user_emb, item_emb,
         jnp.zeros((NPAD - NN, D), jnp.float32)], axis=0)

    npad_e = EDGES_PAD - EDGES
    src_p = jnp.concatenate(
        [edge_src.astype(jnp.int32), jnp.zeros((npad_e,), jnp.int32)])
    dst_p = jnp.concatenate(
        [edge_dst.astype(jnp.int32),
         NN + (jnp.arange(npad_e, dtype=jnp.int32) % (NPAD - NN))])
    val_p = jnp.concatenate([adj_vals, jnp.zeros((npad_e,), jnp.float32)])

    src2 = src_p.reshape(ROWS_PAD, CHUNK)
    dst3 = dst_p.reshape(ROWS_PAD, CHUNK // 16, 16)
    val2 = val_p.reshape(ROWS_PAD, CHUNK)

    msum = jnp.zeros((_CR, _CW), jnp.float32)
    for layer in range(N_LAYERS):
        parts = _sc_layer(ego, src2, dst3, val2)
        ego, msum = _combine(parts, msum, last=(layer == N_LAYERS - 1))

    final = msum.reshape(NPAD, D)
    return final[:NUM_USERS], final[NUM_USERS:NN]
